# Initial kernel scaffold; baseline (speedup 1.0000x reference)
import jax
import jax.numpy as jnp
from jax.experimental import pallas as pl


def kernel(inputs, predicted_field, charge_emb, W_res1, b_res1, W_ef1, b_ef1, W_efp, b_efp, W_ef2, b_ef2, W_m3a, b_m3a, W_m3b, b_m3b, W_m4a, b_m4a, W_m4b, b_m4b, Wi_f, Wh_f, bi_f, bh_f, Wi_r, Wh_r, bi_r, bh_r, W_p1, b_p1, W_p2, b_p2, W_e1, b_e1, W_e2, b_e2):
    raise NotImplementedError("write your pallas kernel here")


# speedup vs baseline: 6.8800x; 6.8800x over previous
import functools

import jax
import jax.numpy as jnp
import numpy as np
from jax.experimental import pallas as pl
from jax.experimental.pallas import tpu as pltpu

N = 32
T = 32
H = 128
SB = 8
EBLK = 256


def _elu(x):
    return jnp.where(x > 0, x, jnp.exp(x) - 1.0)


def _edge_filter_kernel(x_ref, ce_ref, wef1_ref, bef1_ref, wefp_ref, befp_ref,
                        wef2_ref, bef2_ref,
                        eo_ref, nsum_ref, rsum_ref):
    sb = pl.program_id(1)

    x = x_ref[0]
    ce = ce_ref[0]
    pos = x[..., 0:2]
    vel = x[..., 2:4]

    xs = x_ref[0, pl.ds(sb * SB, SB)]
    ces = ce_ref[0, pl.ds(sb * SB, SB)]
    ps = xs[..., 0:2]
    vs = xs[..., 2:4]

    ps_b = jnp.broadcast_to(ps[:, None], (SB, N, T, 2))
    vs_b = jnp.broadcast_to(vs[:, None], (SB, N, T, 2))
    pr_b = jnp.broadcast_to(pos[None], (SB, N, T, 2))
    vr_b = jnp.broadcast_to(vel[None], (SB, N, T, 2))
    x_r = jnp.broadcast_to(x[None], (SB, N, T, 8))
    ce_r = jnp.broadcast_to(ce[None], (SB, N, T, 16))
    ce_s = jnp.broadcast_to(ces[:, None], (SB, N, T, 16))

    rp = ps_b - pr_b
    rv = vs_b - vr_b
    orient = rp[..., 0:1] * rv[..., 1:2] - rp[..., 1:2] * rv[..., 0:1]

    rel_sr = jnp.concatenate([rp, rv, vs_b, vr_b, orient], -1)
    edge_attr = jnp.concatenate(
        [rp, rv, vs_b, vr_b, orient,
         -rp, -rv, vr_b, vs_b, orient,
         x_r, rp, ce_r, ce_s], -1)

    rows = SB * N * T
    h = _elu(edge_attr.reshape(rows, 60) @ wef1_ref[...] + bef1_ref[...])
    wefp = wefp_ref[...]
    g_pre = (rp[..., 0:1] * wefp[0] + rp[..., 1:2] * wefp[1]
             + orient * wefp[2] + befp_ref[0])
    g = jnp.tanh(g_pre.reshape(rows, H))
    eo = _elu((h * g) @ wef2_ref[...] + bef2_ref[...])
    eo = eo.reshape(SB, N, T, H)

    s_ids = sb * SB + jax.lax.broadcasted_iota(jnp.int32, (SB, N), 0)
    r_ids = jax.lax.broadcasted_iota(jnp.int32, (SB, N), 1)
    mask = (s_ids != r_ids).astype(jnp.float32)[:, :, None, None]
    eo = eo * mask
    eo_ref[0] = eo.reshape(SB * N, T, H)

    ns = jnp.sum(eo, axis=0)
    rs = jnp.sum(rel_sr * mask, axis=0)

    @pl.when(sb == 0)
    def _init():
        nsum_ref[0] = ns
        rsum_ref[0] = rs

    @pl.when(sb != 0)
    def _acc():
        nsum_ref[0] += ns
        rsum_ref[0] += rs


def _edge_rnn_kernel(eo_ref, nsum_ref, rsum_ref, x_ref, ce_ref,
                     wres_ref, bres_ref, w3a_ref, b3a_ref, w3b_ref, b3b_ref,
                     w4a_ref, b4a_ref, w4b_ref, b4b_ref,
                     wif_ref, whf_ref, bif_ref, bhf_ref,
                     wir_ref, whr_ref, bir_ref, bhr_ref,
                     wp1_ref, bp1_ref, wp2_ref, bp2_ref,
                     we1_ref, be1_ref, we2_ref, be2_ref,
                     prior_ref, enc_ref, ht_ref,
                     e_s, fwd_s, rev_s, vs_s):
    eb = pl.program_id(1)
    SBLK = EBLK // N

    x = x_ref[0]
    ce = ce_ref[0]
    pos = x[..., 0:2]
    inc = rsum_ref[0] * (1.0 / (N - 1))
    cat = jnp.concatenate([x, inc, pos, ce], -1)
    res = cat.reshape(N * T, 35) @ wres_ref[...] + bres_ref[...]
    node = nsum_ref[0].reshape(N * T, H) * (1.0 / (N - 1)) + res
    node = _elu(node @ w3a_ref[...] + b3a_ref[...])
    node = _elu(node @ w3b_ref[...] + b3b_ref[...])

    w4a = w4a_ref[...]
    vs_s[...] = (node @ w4a[0:H]).reshape(N, T, H)
    v_s = vs_s[pl.ds(eb * SBLK, SBLK)].reshape(SBLK, 1, T, H)
    v_r = (node @ w4a[H:2 * H]).reshape(1, N, T, H)
    eo = eo_ref[0]
    v_e = (eo.reshape(EBLK * T, H) @ w4a[2 * H:3 * H]).reshape(SBLK, N, T, H)
    m4h = _elu(v_s + v_r + v_e + b4a_ref[0])
    e_val = _elu(m4h.reshape(EBLK * T, H) @ w4b_ref[...] + b4b_ref[...])
    e_s[...] = jnp.transpose(e_val.reshape(SBLK, N, T, H), (2, 0, 1, 3)
                             ).reshape(T, EBLK, H)

    def gru_step(t, h, wi, bi, wh, bh):
        xt = e_s[pl.ds(t, 1)].reshape(EBLK, H)
        gi = xt @ wi + bi
        gh = h @ wh + bh
        r = jax.nn.sigmoid(gi[:, 0:H] + gh[:, 0:H])
        z = jax.nn.sigmoid(gi[:, H:2 * H] + gh[:, H:2 * H])
        n = jnp.tanh(gi[:, 2 * H:3 * H] + r * gh[:, 2 * H:3 * H])
        return (1.0 - z) * n + z * h

    wif, bif, whf, bhf = wif_ref[...], bif_ref[...], whf_ref[...], bhf_ref[...]

    def fwd_body(t, h):
        h = gru_step(t, h, wif, bif, whf, bhf)
        fwd_s[pl.ds(t, 1)] = h[None]
        return h

    h0 = jnp.zeros((EBLK, H), jnp.float32)
    h_t = jax.lax.fori_loop(0, T, fwd_body, h0)
    ht_ref[0] = h_t

    wir, bir, whr, bhr = wir_ref[...], bir_ref[...], whr_ref[...], bhr_ref[...]

    def rev_body(k, h):
        t = T - 1 - k
        h = gru_step(t, h, wir, bir, whr, bhr)
        rev_s[pl.ds(t, 1)] = h[None]
        return h

    jax.lax.fori_loop(0, T, rev_body, h0)

    fx = fwd_s[...].reshape(T * EBLK, H)
    ph = _elu(fx @ wp1_ref[...] + bp1_ref[...])
    prior_ref[0] = (ph @ wp2_ref[...] + bp2_ref[...]).reshape(T, EBLK, 4)

    comb = jnp.concatenate([fx, rev_s[...].reshape(T * EBLK, H)], -1)
    eh = _elu(comb @ we1_ref[...] + be1_ref[...])
    enc_ref[0] = (eh @ we2_ref[...] + be2_ref[...]).reshape(T, EBLK, 4)


_REAL_EDGES = np.array([d for d in range(N * N) if d // N != d % N])


def kernel(inputs, predicted_field, charge_emb, W_res1, b_res1, W_ef1, b_ef1,
           W_efp, b_efp, W_ef2, b_ef2, W_m3a, b_m3a, W_m3b, b_m3b, W_m4a,
           b_m4a, W_m4b, b_m4b, Wi_f, Wh_f, bi_f, bh_f, Wi_r, Wh_r, bi_r,
           bh_r, W_p1, b_p1, W_p2, b_p2, W_e1, b_e1, W_e2, b_e2):
    B = inputs.shape[0]
    x = jnp.transpose(inputs, (0, 2, 1, 3))

    r2 = lambda b: b.reshape(1, -1)

    eo, nsum, rsum = pl.pallas_call(
        _edge_filter_kernel,
        grid=(B, N // SB),
        in_specs=[
            pl.BlockSpec((1, N, T, 8), lambda b, s: (b, 0, 0, 0)),
            pl.BlockSpec((1, N, T, 16), lambda b, s: (b, 0, 0, 0)),
            pl.BlockSpec((60, H), lambda b, s: (0, 0)),
            pl.BlockSpec((1, H), lambda b, s: (0, 0)),
            pl.BlockSpec((3, H), lambda b, s: (0, 0)),
            pl.BlockSpec((1, H), lambda b, s: (0, 0)),
            pl.BlockSpec((H, H), lambda b, s: (0, 0)),
            pl.BlockSpec((1, H), lambda b, s: (0, 0)),
        ],
        out_specs=[
            pl.BlockSpec((1, SB * N, T, H), lambda b, s: (b, s, 0, 0)),
            pl.BlockSpec((1, N, T, H), lambda b, s: (b, 0, 0, 0)),
            pl.BlockSpec((1, N, T, 9), lambda b, s: (b, 0, 0, 0)),
        ],
        out_shape=[
            jax.ShapeDtypeStruct((B, N * N, T, H), jnp.float32),
            jax.ShapeDtypeStruct((B, N, T, H), jnp.float32),
            jax.ShapeDtypeStruct((B, N, T, 9), jnp.float32),
        ],
    )(x, charge_emb, W_ef1.T, r2(b_ef1), W_efp.T, r2(b_efp), W_ef2.T,
      r2(b_ef2))

    prior_d, enc_d, ht_d = pl.pallas_call(
        _edge_rnn_kernel,
        grid=(B, (N * N) // EBLK),
        in_specs=[
            pl.BlockSpec((1, EBLK, T, H), lambda b, e: (b, e, 0, 0)),
            pl.BlockSpec((1, N, T, H), lambda b, e: (b, 0, 0, 0)),
            pl.BlockSpec((1, N, T, 9), lambda b, e: (b, 0, 0, 0)),
            pl.BlockSpec((1, N, T, 8), lambda b, e: (b, 0, 0, 0)),
            pl.BlockSpec((1, N, T, 16), lambda b, e: (b, 0, 0, 0)),
            pl.BlockSpec((35, H), lambda b, e: (0, 0)),
            pl.BlockSpec((1, H), lambda b, e: (0, 0)),
            pl.BlockSpec((H, H), lambda b, e: (0, 0)),
            pl.BlockSpec((1, H), lambda b, e: (0, 0)),
            pl.BlockSpec((H, H), lambda b, e: (0, 0)),
            pl.BlockSpec((1, H), lambda b, e: (0, 0)),
            pl.BlockSpec((3 * H, H), lambda b, e: (0, 0)),
            pl.BlockSpec((1, H), lambda b, e: (0, 0)),
            pl.BlockSpec((H, H), lambda b, e: (0, 0)),
            pl.BlockSpec((1, H), lambda b, e: (0, 0)),
            pl.BlockSpec((H, 3 * H), lambda b, e: (0, 0)),
            pl.BlockSpec((H, 3 * H), lambda b, e: (0, 0)),
            pl.BlockSpec((1, 3 * H), lambda b, e: (0, 0)),
            pl.BlockSpec((1, 3 * H), lambda b, e: (0, 0)),
            pl.BlockSpec((H, 3 * H), lambda b, e: (0, 0)),
            pl.BlockSpec((H, 3 * H), lambda b, e: (0, 0)),
            pl.BlockSpec((1, 3 * H), lambda b, e: (0, 0)),
            pl.BlockSpec((1, 3 * H), lambda b, e: (0, 0)),
            pl.BlockSpec((H, H), lambda b, e: (0, 0)),
            pl.BlockSpec((1, H), lambda b, e: (0, 0)),
            pl.BlockSpec((H, 4), lambda b, e: (0, 0)),
            pl.BlockSpec((1, 4), lambda b, e: (0, 0)),
            pl.BlockSpec((2 * H, 2 * H), lambda b, e: (0, 0)),
            pl.BlockSpec((1, 2 * H), lambda b, e: (0, 0)),
            pl.BlockSpec((2 * H, 4), lambda b, e: (0, 0)),
            pl.BlockSpec((1, 4), lambda b, e: (0, 0)),
        ],
        out_specs=[
            pl.BlockSpec((1, T, EBLK, 4), lambda b, e: (b, 0, e, 0)),
            pl.BlockSpec((1, T, EBLK, 4), lambda b, e: (b, 0, e, 0)),
            pl.BlockSpec((1, EBLK, H), lambda b, e: (b, e, 0)),
        ],
        out_shape=[
            jax.ShapeDtypeStruct((B, T, N * N, 4), jnp.float32),
            jax.ShapeDtypeStruct((B, T, N * N, 4), jnp.float32),
            jax.ShapeDtypeStruct((B, N * N, H), jnp.float32),
        ],
        scratch_shapes=[
            pltpu.VMEM((T, EBLK, H), jnp.float32),
            pltpu.VMEM((T, EBLK, H), jnp.float32),
            pltpu.VMEM((T, EBLK, H), jnp.float32),
            pltpu.VMEM((N, T, H), jnp.float32),
        ],
    )(eo, nsum, rsum, x, charge_emb,
      W_res1.T, r2(b_res1), W_m3a.T, r2(b_m3a), W_m3b.T, r2(b_m3b),
      W_m4a.T, r2(b_m4a), W_m4b.T, r2(b_m4b),
      Wi_f.T, Wh_f.T, r2(bi_f), r2(bh_f),
      Wi_r.T, Wh_r.T, r2(bi_r), r2(bh_r),
      W_p1.T, r2(b_p1), W_p2.T, r2(b_p2),
      W_e1.T, r2(b_e1), W_e2.T, r2(b_e2))

    idx = jnp.asarray(_REAL_EDGES)
    prior_result = prior_d[:, :, idx, :]
    encoder_result = enc_d[:, :, idx, :]
    prior_state = ht_d[:, idx, :].reshape(1, B * (N * N - N), H)
    return (prior_result, encoder_result, prior_state)


# speedup vs baseline: 8.4554x; 1.2290x over previous
import jax
import jax.numpy as jnp
import numpy as np
from jax.experimental import pallas as pl
from jax.experimental.pallas import tpu as pltpu

N = 32
T = 32
H = 128
SB = 8
EBLK = 256
HCH = 8


def _elu(x):
    return jnp.where(x > 0, x, jnp.exp(x) - 1.0)


def _cross(ax, ay, bx, by):
    return ax * by - ay * bx


def _edge_filter_kernel(x_ref, ce_ref, wsg_ref, wrg_ref, brow_ref, wo_ref,
                        wp2_ref, wef2_ref, bef2_ref,
                        eo_ref, nsum_ref):
    sb = pl.program_id(1)

    x = x_ref[0]
    ce = ce_ref[0]
    c_full = _cross(x[..., 0:1], x[..., 1:2], x[..., 2:3], x[..., 3:4])
    nf = jnp.concatenate([x, ce, c_full], -1)

    xs = x_ref[0, pl.ds(sb * SB, SB)]
    ces = ce_ref[0, pl.ds(sb * SB, SB)]
    c_s = _cross(xs[..., 0:1], xs[..., 1:2], xs[..., 2:3], xs[..., 3:4])
    nfs = jnp.concatenate([xs, ces, c_s], -1)

    S2 = nfs.reshape(SB * T, 25) @ wsg_ref[...]
    R2 = nf.reshape(N * T, 25) @ wrg_ref[...] + brow_ref[...]
    S2 = S2.reshape(SB, 1, T, 2 * H)
    R2 = R2.reshape(1, N, T, 2 * H)

    psx, psy = xs[..., 0:1], xs[..., 1:2]
    vsx, vsy = xs[..., 2:3], xs[..., 3:4]
    prx, pry = x[..., 0:1], x[..., 1:2]
    vrx, vry = x[..., 2:3], x[..., 3:4]
    Bi = (psx[:, None] * vry[None] - psy[:, None] * vrx[None]
          + prx[None] * vsy[:, None] - pry[None] * vsx[:, None])

    h = _elu(S2[..., 0:H] + R2[..., 0:H] - Bi * wo_ref[0])
    g = jnp.tanh(S2[..., H:2 * H] + R2[..., H:2 * H] - Bi * wp2_ref[0])

    rows = SB * N * T
    eo = _elu((h * g).reshape(rows, H) @ wef2_ref[...] + bef2_ref[...])
    eo = eo.reshape(SB, N, T, H)

    s_ids = sb * SB + jax.lax.broadcasted_iota(jnp.int32, (SB, N), 0)
    r_ids = jax.lax.broadcasted_iota(jnp.int32, (SB, N), 1)
    mask = (s_ids != r_ids).astype(jnp.float32)[:, :, None, None]
    eo = eo * mask
    eo_ref[0] = eo.reshape(SB * N, T, H)

    ns = jnp.sum(eo, axis=0)

    @pl.when(sb == 0)
    def _init():
        nsum_ref[0] = ns

    @pl.when(sb != 0)
    def _acc():
        nsum_ref[0] += ns


def _edge_rnn_kernel(eo_ref, nsum_ref, x_ref, ce_ref,
                     wres_ref, bres_ref, w3a_ref, b3a_ref, w3b_ref, b3b_ref,
                     w4a_ref, b4a_ref, w4b_ref, b4b_ref,
                     wif_ref, whf_ref, bif_ref, bhf_ref,
                     wir_ref, whr_ref, bir_ref, bhr_ref,
                     wp1_ref, bp1_ref, wp2_ref, bp2_ref,
                     we1_ref, be1_ref, we2_ref, be2_ref,
                     prior_ref, enc_ref, ht_ref,
                     e_s, gif_s, fwd_s, rev_s, vs_s):
    eb = pl.program_id(1)
    SBLK = EBLK // N

    x = x_ref[0]
    ce = ce_ref[0]
    p = x[..., 0:2]
    v = x[..., 2:4]
    c = _cross(x[..., 0:1], x[..., 1:2], x[..., 2:3], x[..., 3:4])
    p_tot = jnp.sum(p, axis=0, keepdims=True)
    v_tot = jnp.sum(v, axis=0, keepdims=True)
    c_tot = jnp.sum(c, axis=0, keepdims=True)
    sum_o = (c_tot + 32.0 * c
             - _cross(p_tot[..., 0:1], p_tot[..., 1:2],
                      x[..., 2:3], x[..., 3:4])
             - _cross(x[..., 0:1], x[..., 1:2],
                      v_tot[..., 0:1], v_tot[..., 1:2]))
    inc = jnp.concatenate(
        [p_tot - 32.0 * p, v_tot - 32.0 * v, v_tot - v,
         31.0 * v, sum_o], -1) * (1.0 / (N - 1))

    cat = jnp.concatenate([x, inc, p, ce], -1)
    res = cat.reshape(N * T, 35) @ wres_ref[...] + bres_ref[...]
    node = nsum_ref[0].reshape(N * T, H) * (1.0 / (N - 1)) + res
    node = _elu(node @ w3a_ref[...] + b3a_ref[...])
    node = _elu(node @ w3b_ref[...] + b3b_ref[...])

    w4a = w4a_ref[...]
    vs_s[...] = (node @ w4a[0:H]).reshape(N, T, H)
    v_s = vs_s[pl.ds(eb * SBLK, SBLK)].reshape(SBLK, 1, T, H)
    v_r = (node @ w4a[H:2 * H]).reshape(1, N, T, H)
    eo = eo_ref[0]
    v_e = (eo.reshape(EBLK * T, H) @ w4a[2 * H:3 * H]).reshape(SBLK, N, T, H)
    m4h = _elu(v_s + v_r + v_e + b4a_ref[0])
    e_val = _elu(m4h.reshape(EBLK * T, H) @ w4b_ref[...] + b4b_ref[...])
    e_s[...] = jnp.transpose(e_val.reshape(SBLK, N, T, H), (2, 0, 1, 3)
                             ).reshape(T, EBLK, H)

    e2 = e_s[...].reshape(T * EBLK, H)
    gif_s[...] = (e2 @ wif_ref[...] + bif_ref[...]).reshape(T, EBLK, 3 * H)

    whf, bhf = whf_ref[...], bhf_ref[...]
    whr, bhr = whr_ref[...], bhr_ref[...]
    wir, bir = wir_ref[...], bir_ref[...]

    def gru_update(gi, h, wh, bh):
        gh = h @ wh + bh
        r = jax.nn.sigmoid(gi[:, 0:H] + gh[:, 0:H])
        z = jax.nn.sigmoid(gi[:, H:2 * H] + gh[:, H:2 * H])
        n = jnp.tanh(gi[:, 2 * H:3 * H] + r * gh[:, 2 * H:3 * H])
        return n + z * (h - n)

    def body(k, carry):
        hf, hr = carry
        tr = T - 1 - k
        gif = gif_s[pl.ds(k, 1)].reshape(EBLK, 3 * H)
        hf = gru_update(gif, hf, whf, bhf)
        fwd_s[pl.ds(k, 1)] = hf[None]
        gir = e_s[pl.ds(tr, 1)].reshape(EBLK, H) @ wir + bir
        hr = gru_update(gir, hr, whr, bhr)
        rev_s[pl.ds(tr, 1)] = hr[None]
        return hf, hr

    h0 = jnp.zeros((EBLK, H), jnp.float32)
    h_t, _ = jax.lax.fori_loop(0, T, body, (h0, h0))
    ht_ref[0] = h_t

    for cst in range(0, T, HCH):
        fx = fwd_s[cst:cst + HCH].reshape(HCH * EBLK, H)
        ph = _elu(fx @ wp1_ref[...] + bp1_ref[...])
        prior_ref[0, cst:cst + HCH] = (
            ph @ wp2_ref[...] + bp2_ref[...]).reshape(HCH, EBLK, 4)
        rx = rev_s[cst:cst + HCH].reshape(HCH * EBLK, H)
        comb = jnp.concatenate([fx, rx], -1)
        eh = _elu(comb @ we1_ref[...] + be1_ref[...])
        enc_ref[0, cst:cst + HCH] = (
            eh @ we2_ref[...] + be2_ref[...]).reshape(HCH, EBLK, 4)


_REAL_EDGES = np.array([d for d in range(N * N) if d // N != d % N])


def kernel(inputs, predicted_field, charge_emb, W_res1, b_res1, W_ef1, b_ef1,
           W_efp, b_efp, W_ef2, b_ef2, W_m3a, b_m3a, W_m3b, b_m3b, W_m4a,
           b_m4a, W_m4b, b_m4b, Wi_f, Wh_f, bi_f, bh_f, Wi_r, Wh_r, bi_r,
           bh_r, W_p1, b_p1, W_p2, b_p2, W_e1, b_e1, W_e2, b_e2):
    B = inputs.shape[0]
    x = jnp.transpose(inputs, (0, 2, 1, 3))

    r2 = lambda b: b.reshape(1, -1)

    W1 = W_ef1.T
    Wp = W_efp.T
    w_o = W1[8] + W1[17]
    z4 = jnp.zeros((4, H), jnp.float32)
    z22 = jnp.zeros((22, H), jnp.float32)
    wrp = W1[0:2] - W1[9:11] + W1[26:28]
    wrv = W1[2:4] - W1[11:13]
    WS = jnp.concatenate(
        [wrp, wrv + W1[4:6] + W1[15:17], z4, W1[44:60], w_o[None]], 0)
    WR = jnp.concatenate(
        [jnp.concatenate([W1[18:20] - wrp,
                          W1[20:22] - wrv + W1[6:8] + W1[13:15],
                          W1[22:26]], 0),
         W1[28:44], w_o[None]], 0)
    WGS = jnp.concatenate([Wp[0:2], z22, Wp[2][None]], 0)
    WGR = jnp.concatenate([-Wp[0:2], z22, Wp[2][None]], 0)
    WSG = jnp.concatenate([WS, WGS], 1)
    WRG = jnp.concatenate([WR, WGR], 1)
    brow = jnp.concatenate([b_ef1, b_efp], 0).reshape(1, 2 * H)

    eo, nsum = pl.pallas_call(
        _edge_filter_kernel,
        grid=(B, N // SB),
        in_specs=[
            pl.BlockSpec((1, N, T, 8), lambda b, s: (b, 0, 0, 0)),
            pl.BlockSpec((1, N, T, 16), lambda b, s: (b, 0, 0, 0)),
            pl.BlockSpec((25, 2 * H), lambda b, s: (0, 0)),
            pl.BlockSpec((25, 2 * H), lambda b, s: (0, 0)),
            pl.BlockSpec((1, 2 * H), lambda b, s: (0, 0)),
            pl.BlockSpec((1, H), lambda b, s: (0, 0)),
            pl.BlockSpec((1, H), lambda b, s: (0, 0)),
            pl.BlockSpec((H, H), lambda b, s: (0, 0)),
            pl.BlockSpec((1, H), lambda b, s: (0, 0)),
        ],
        out_specs=[
            pl.BlockSpec((1, SB * N, T, H), lambda b, s: (b, s, 0, 0)),
            pl.BlockSpec((1, N, T, H), lambda b, s: (b, 0, 0, 0)),
        ],
        out_shape=[
            jax.ShapeDtypeStruct((B, N * N, T, H), jnp.float32),
            jax.ShapeDtypeStruct((B, N, T, H), jnp.float32),
        ],
    )(x, charge_emb, WSG, WRG, brow, w_o.reshape(1, H), Wp[2].reshape(1, H),
      W_ef2.T, r2(b_ef2))

    prior_d, enc_d, ht_d = pl.pallas_call(
        _edge_rnn_kernel,
        grid=(B, (N * N) // EBLK),
        in_specs=[
            pl.BlockSpec((1, EBLK, T, H), lambda b, e: (b, e, 0, 0)),
            pl.BlockSpec((1, N, T, H), lambda b, e: (b, 0, 0, 0)),
            pl.BlockSpec((1, N, T, 8), lambda b, e: (b, 0, 0, 0)),
            pl.BlockSpec((1, N, T, 16), lambda b, e: (b, 0, 0, 0)),
            pl.BlockSpec((35, H), lambda b, e: (0, 0)),
            pl.BlockSpec((1, H), lambda b, e: (0, 0)),
            pl.BlockSpec((H, H), lambda b, e: (0, 0)),
            pl.BlockSpec((1, H), lambda b, e: (0, 0)),
            pl.BlockSpec((H, H), lambda b, e: (0, 0)),
            pl.BlockSpec((1, H), lambda b, e: (0, 0)),
            pl.BlockSpec((3 * H, H), lambda b, e: (0, 0)),
            pl.BlockSpec((1, H), lambda b, e: (0, 0)),
            pl.BlockSpec((H, H), lambda b, e: (0, 0)),
            pl.BlockSpec((1, H), lambda b, e: (0, 0)),
            pl.BlockSpec((H, 3 * H), lambda b, e: (0, 0)),
            pl.BlockSpec((H, 3 * H), lambda b, e: (0, 0)),
            pl.BlockSpec((1, 3 * H), lambda b, e: (0, 0)),
            pl.BlockSpec((1, 3 * H), lambda b, e: (0, 0)),
            pl.BlockSpec((H, 3 * H), lambda b, e: (0, 0)),
            pl.BlockSpec((H, 3 * H), lambda b, e: (0, 0)),
            pl.BlockSpec((1, 3 * H), lambda b, e: (0, 0)),
            pl.BlockSpec((1, 3 * H), lambda b, e: (0, 0)),
            pl.BlockSpec((H, H), lambda b, e: (0, 0)),
            pl.BlockSpec((1, H), lambda b, e: (0, 0)),
            pl.BlockSpec((H, 4), lambda b, e: (0, 0)),
            pl.BlockSpec((1, 4), lambda b, e: (0, 0)),
            pl.BlockSpec((2 * H, 2 * H), lambda b, e: (0, 0)),
            pl.BlockSpec((1, 2 * H), lambda b, e: (0, 0)),
            pl.BlockSpec((2 * H, 4), lambda b, e: (0, 0)),
            pl.BlockSpec((1, 4), lambda b, e: (0, 0)),
        ],
        out_specs=[
            pl.BlockSpec((1, T, EBLK, 4), lambda b, e: (b, 0, e, 0)),
            pl.BlockSpec((1, T, EBLK, 4), lambda b, e: (b, 0, e, 0)),
            pl.BlockSpec((1, EBLK, H), lambda b, e: (b, e, 0)),
        ],
        out_shape=[
            jax.ShapeDtypeStruct((B, T, N * N, 4), jnp.float32),
            jax.ShapeDtypeStruct((B, T, N * N, 4), jnp.float32),
            jax.ShapeDtypeStruct((B, N * N, H), jnp.float32),
        ],
        scratch_shapes=[
            pltpu.VMEM((T, EBLK, H), jnp.float32),
            pltpu.VMEM((T, EBLK, 3 * H), jnp.float32),
            pltpu.VMEM((T, EBLK, H), jnp.float32),
            pltpu.VMEM((T, EBLK, H), jnp.float32),
            pltpu.VMEM((N, T, H), jnp.float32),
        ],
    )(eo, nsum, x, charge_emb,
      W_res1.T, r2(b_res1), W_m3a.T, r2(b_m3a), W_m3b.T, r2(b_m3b),
      W_m4a.T, r2(b_m4a), W_m4b.T, r2(b_m4b),
      Wi_f.T, Wh_f.T, r2(bi_f), r2(bh_f),
      Wi_r.T, Wh_r.T, r2(bi_r), r2(bh_r),
      W_p1.T, r2(b_p1), W_p2.T, r2(b_p2),
      W_e1.T, r2(b_e1), W_e2.T, r2(b_e2))

    idx = jnp.asarray(_REAL_EDGES)
    prior_result = prior_d[:, :, idx, :]
    encoder_result = enc_d[:, :, idx, :]
    prior_state = ht_d[:, idx, :].reshape(1, B * (N * N - N), H)
    return (prior_result, encoder_result, prior_state)
